# baseline (speedup 1.0000x reference)
import functools

import jax
import jax.numpy as jnp
from jax import lax
from jax.experimental import pallas as pl
from jax.experimental.pallas import tpu as pltpu
from jax.experimental.pallas import tpu_sc as plsc

B = 1024
S = 49999
K = 128
NC, NS, L = 2, 16, 16
NW = NC * NS
RW = B // NW


def _make_gather():
    mesh = plsc.VectorSubcoreMesh(core_axis_name="c", subcore_axis_name="s")

    @functools.partial(
        pl.kernel,
        mesh=mesh,
        out_type=jax.ShapeDtypeStruct((B * K,), jnp.float32),
        scratch_types=[
            pltpu.VMEM((K,), jnp.int32),
            pltpu.VMEM((RW * K,), jnp.int32),
            pltpu.VMEM((RW * K,), jnp.float32),
            pltpu.SemaphoreType.DMA,
        ],
    )
    def gather_kernel(state_hbm, obs_hbm, out_hbm, obs_v, idx_v, vals_v, sem):
        wid = lax.axis_index("s") * NC + lax.axis_index("c")
        base_row = wid * RW
        pltpu.sync_copy(obs_hbm, obs_v)

        def idx_body(r, carry):
            rb = (base_row + r) * S
            for kk in range(K // L):
                idx_v[pl.ds(r * K + kk * L, L)] = obs_v[pl.ds(kk * L, L)] + rb
            return carry

        lax.fori_loop(0, RW, idx_body, 0)

        def gather_body(r, carry):
            src = state_hbm.at[idx_v.at[pl.ds(r * K, K)]]
            pltpu.async_copy(src, vals_v.at[pl.ds(r * K, K)], sem).wait()
            return carry

        lax.fori_loop(0, RW, gather_body, 0)

        pltpu.sync_copy(vals_v, out_hbm.at[pl.ds(base_row * K, RW * K)])

    return gather_kernel


_gather = _make_gather()


def kernel(state, obs_tensor):
    out_flat = _gather(state.reshape(-1), obs_tensor)
    return out_flat.reshape(B, K)


# speedup vs baseline: 1.0105x; 1.0105x over previous
import functools

import jax
import jax.numpy as jnp
from jax import lax
from jax.experimental import pallas as pl
from jax.experimental.pallas import tpu as pltpu
from jax.experimental.pallas import tpu_sc as plsc

B = 1024
S = 49999
K = 128
NC, NS, L = 2, 16, 16
NW = NC * NS
RW = B // NW


def _make_gather():
    mesh = plsc.VectorSubcoreMesh(core_axis_name="c", subcore_axis_name="s")

    @functools.partial(
        pl.kernel,
        mesh=mesh,
        out_type=jax.ShapeDtypeStruct((B * K,), jnp.float32),
        scratch_types=[
            pltpu.VMEM((K,), jnp.int32),
            pltpu.VMEM((RW * K,), jnp.int32),
            pltpu.VMEM((RW * K,), jnp.float32),
            pltpu.SemaphoreType.DMA,
        ],
    )
    def gather_kernel(state_hbm, obs_hbm, out_hbm, obs_v, idx_v, vals_v, sem):
        wid = lax.axis_index("s") * NC + lax.axis_index("c")
        base_row = wid * RW
        pltpu.sync_copy(obs_hbm, obs_v)

        def idx_body(r, carry):
            rb = (base_row + r) * S
            for kk in range(K // L):
                idx_v[pl.ds(r * K + kk * L, L)] = obs_v[pl.ds(kk * L, L)] + rb
            return carry

        lax.fori_loop(0, RW, idx_body, 0)

        def fire_body(r, carry):
            src = state_hbm.at[idx_v.at[pl.ds(r * K, K)]]
            pltpu.async_copy(src, vals_v.at[pl.ds(r * K, K)], sem)
            return carry

        lax.fori_loop(0, RW, fire_body, 0)

        def drain_body(r, carry):
            src = state_hbm.at[idx_v.at[pl.ds(r * K, K)]]
            pltpu.make_async_copy(src, vals_v.at[pl.ds(r * K, K)], sem).wait()
            return carry

        lax.fori_loop(0, RW, drain_body, 0)

        pltpu.sync_copy(vals_v, out_hbm.at[pl.ds(base_row * K, RW * K)])

    return gather_kernel


_gather = _make_gather()


def kernel(state, obs_tensor):
    out_flat = _gather(state.reshape(-1), obs_tensor)
    return out_flat.reshape(B, K)


# speedup vs baseline: 8.9654x; 8.8719x over previous
import jax
import jax.numpy as jnp
from jax import lax
from jax.experimental import pallas as pl
from jax.experimental.pallas import tpu as pltpu

B = 1024
S = 49999
K = 128


def _gather_body(obs_sm, state_blk, out_ref):
    j = pl.program_id(0)
    col = obs_sm[j]
    lane_in = col - (col // K) * K
    lane = lax.broadcasted_iota(jnp.int32, (1, K), 1)
    sel_l = (lane == lane_in).astype(jnp.float32)
    sel_j = (lane == j).astype(jnp.float32)
    v = jnp.sum(state_blk[...] * sel_l, axis=1, keepdims=True)
    contrib = v * sel_j

    @pl.when(j == 0)
    def _init():
        out_ref[...] = contrib

    @pl.when(j != 0)
    def _acc():
        out_ref[...] += contrib


def kernel(state, obs_tensor):
    grid_spec = pltpu.PrefetchScalarGridSpec(
        num_scalar_prefetch=1,
        grid=(K,),
        in_specs=[
            pl.BlockSpec((B, K), lambda j, obs: (0, obs[j] // K)),
        ],
        out_specs=pl.BlockSpec((B, K), lambda j, obs: (0, 0)),
    )
    return pl.pallas_call(
        _gather_body,
        grid_spec=grid_spec,
        out_shape=jax.ShapeDtypeStruct((B, K), jnp.float32),
    )(obs_tensor, state)


# speedup vs baseline: 11.5651x; 1.2900x over previous
import jax
import jax.numpy as jnp
from jax import lax
from jax.experimental import pallas as pl
from jax.experimental.pallas import tpu as pltpu

B = 1024
S = 49999
K = 128
NBUF = 8


def _gather_body(obs_sm, state_hbm, out_ref, bufs, sems):
    def block_copy(j, slot):
        ct = obs_sm[j] // K
        src = state_hbm.at[:, pl.ds(pl.multiple_of(ct * K, K), K)]
        return pltpu.make_async_copy(src, bufs.at[slot], sems.at[slot])

    for s in range(NBUF):
        block_copy(s, s).start()

    lane = lax.broadcasted_iota(jnp.int32, (1, K), 1)

    def body(j, carry):
        slot = lax.rem(j, NBUF)
        block_copy(j, slot).wait()
        col = obs_sm[j]
        l = lax.rem(col, K)
        rolled = pltpu.roll(bufs[slot], j - l, axis=1)
        out_ref[...] = jnp.where(lane == j, rolled, out_ref[...])

        @pl.when(j + NBUF < K)
        def _refire():
            block_copy(j + NBUF, slot).start()

        return carry

    lax.fori_loop(0, K, body, 0)


def kernel(state, obs_tensor):
    return pl.pallas_call(
        _gather_body,
        grid=(),
        in_specs=[
            pl.BlockSpec(memory_space=pltpu.SMEM),
            pl.BlockSpec(memory_space=pl.ANY),
        ],
        out_specs=pl.BlockSpec(memory_space=pltpu.VMEM),
        out_shape=jax.ShapeDtypeStruct((B, K), jnp.float32),
        scratch_shapes=[
            pltpu.VMEM((NBUF, B, K), jnp.float32),
            pltpu.SemaphoreType.DMA((NBUF,)),
        ],
    )(obs_tensor, state)
